# baseline (speedup 1.0000x reference)
import functools

import jax
import jax.numpy as jnp
from jax import lax
from jax.experimental import pallas as pl
from jax.experimental.pallas import tpu as pltpu
from jax.experimental.pallas import tpu_sc as plsc

DIM = 64
L = 16
NC = 2
NS = 16
NW = NC * NS
TILE = 128
WTILES = 4
WIN = WTILES * TILE
RCHUNK = 128


def _cdiv(a, b):
    return (a + b - 1) // b


@functools.lru_cache(maxsize=None)
def _build_sc(batch, n_entity):
    bpw = batch // NW
    ntiles = _cdiv(n_entity, TILE)
    full_tiles = n_entity // TILE
    tail_base = full_tiles * TILE
    npieces = _cdiv(batch, 8192)
    rcap = 2 * (batch // L) + L
    mesh = plsc.VectorSubcoreMesh(core_axis_name="c", subcore_axis_name="s")

    out_types = (
        jax.ShapeDtypeStruct((2 * batch + NW, 2 * DIM), jnp.float32),
        jax.ShapeDtypeStruct((batch, 2 * DIM), jnp.float32),
    )

    @functools.partial(
        pl.kernel,
        out_type=out_types,
        mesh=mesh,
        compiler_params=pltpu.CompilerParams(needs_layout_passes=False),
        scratch_types=[
            pltpu.VMEM((DIM, WIN), jnp.float32),
            pltpu.VMEM((DIM, WIN), jnp.float32),
            pltpu.VMEM((L, rcap), jnp.int32),
            pltpu.VMEM((8192,), jnp.int32),
            pltpu.VMEM((2, L, 2 * DIM), jnp.float32),
            pltpu.VMEM((2, L), jnp.int32),
            pltpu.VMEM((RCHUNK, 2 * DIM), jnp.float32),
            pltpu.VMEM((bpw,), jnp.int32),
            pltpu.SemaphoreType.DMA,
            pltpu.SemaphoreType.DMA,
            pltpu.SemaphoreType.DMA,
            pltpu.SemaphoreType.DMA,
        ],
    )
    def sc_kernel(h_hbm, r_hbm, t_hbm, et_hbm, rel_hbm, tail_hbm,
                  uv_hbm, rout_hbm,
                  win0, win1, aren, piece, stg, six,
                  rel_v, ridx, sem_w0, sem_w1, sem_s, sem_r):
        wid = lax.axis_index("s") * NC + lax.axis_index("c")
        base = wid * bpw
        lane = lax.iota(jnp.int32, L)
        perms = [lane ^ k for k in (1, 2, 4, 8)]
        dnums = lax.GatherDimensionNumbers(
            offset_dims=(), collapsed_slice_dims=(0,), start_index_map=(0,))

        def lperm(v, p):
            return lax.gather(v, p[:, None], dnums, slice_sizes=(1,),
                              mode=lax.GatherScatterMode.PROMISE_IN_BOUNDS)

        pltpu.sync_copy(r_hbm.at[pl.ds(base, bpw)], ridx)
        for cc in range(bpw // RCHUNK):
            pltpu.async_copy(
                rel_hbm.at[ridx.at[pl.ds(cc * RCHUNK, RCHUNK)]], rel_v,
                sem_r).wait()
            pltpu.sync_copy(
                rel_v, rout_hbm.at[pl.ds(base + cc * RCHUNK, RCHUNK), :])

        t0 = (wid * ntiles) // NW
        t1 = ((wid + 1) * ntiles) // NW
        t1n = jnp.minimum(t1, jnp.int32(full_tiles))
        has_tail = jnp.where(t1 > t1n, jnp.int32(1), jnp.int32(0))
        nwin = (t1n - t0 + (WTILES - 1)) // WTILES + has_tail

        UNROLL = 4

        def scan_piece(p, cnt, src_hbm, tbl_bit):
            pltpu.sync_copy(src_hbm.at[pl.ds(p * 8192, 8192)], piece)

            def vstep(i, cnt):
                for s in range(UNROLL):
                    ii = i * UNROLL + s
                    v = piece[pl.ds(pl.multiple_of(ii * L, L), L)]
                    tl = v >> 7
                    m = (tl >= t0) & (tl < t1)
                    is_tl = tl >= jnp.int32(full_tiles)
                    wv = jnp.where(is_tl, nwin - 1, (tl - t0) >> 2)
                    tcv = jnp.minimum(t0 + ((tl - t0) >> 2) * WTILES,
                                      t1n - WTILES)
                    colv = jnp.where(is_tl, v - jnp.int32(tail_base),
                                     v - tcv * TILE)
                    pos = jnp.int32(p * 8192) + ii * L + lane
                    entry = (wv << 24) | (colv << 15) | tbl_bit | pos
                    entry = jnp.where(m, entry, jnp.int32(63 << 24))
                    dest = jnp.where(m, cnt, jnp.int32(rcap - L))
                    plsc.store_scatter(aren, [lane, dest], entry)
                    cnt = cnt + jnp.where(m, 1, 0)
                return cnt

            return lax.fori_loop(0, 8192 // (L * UNROLL), vstep, cnt)

        sentinel = jnp.broadcast_to(jnp.int32(63 << 24), (L,))

        def ainit(g, c):
            for j in range(L):
                aren[j, pl.ds(pl.multiple_of(g * L, L), L)] = sentinel
            return c

        lax.fori_loop(0, rcap // L, ainit, 0)

        cnt = jnp.zeros((L,), jnp.int32)
        for p in range(npieces):
            cnt = scan_piece(p, cnt, h_hbm, jnp.int32(0))
        for p in range(npieces):
            cnt = scan_piece(p, cnt, t_hbm, jnp.int32(1 << 14))

        trash = jnp.broadcast_to(jnp.int32(2 * batch), (L,)) + wid
        wins = (win0, win1)
        wsems = (sem_w0, sem_w1)

        def issue(w, buf, sem):
            is_tail = (w == nwin - 1) & (has_tail == 1)

            @pl.when(jnp.logical_not(is_tail))
            def _():
                tc = jnp.minimum(t0 + w * WTILES, t1n - WTILES)
                cb = pl.multiple_of(tc * TILE, TILE)
                pltpu.async_copy(et_hbm.at[:, pl.ds(cb, WIN)], buf, sem)

            @pl.when(is_tail)
            def _():
                pltpu.async_copy(tail_hbm, buf.at[:, pl.ds(0, TILE)], sem)

        def drain(w, buf, sem):
            is_tail = (w == nwin - 1) & (has_tail == 1)

            @pl.when(jnp.logical_not(is_tail))
            def _():
                pltpu.make_async_copy(
                    et_hbm.at[:, pl.ds(0, WIN)], buf, sem).wait()

            @pl.when(is_tail)
            def _():
                pltpu.make_async_copy(
                    tail_hbm, buf.at[:, pl.ds(0, TILE)], sem).wait()

        issue(jnp.int32(0), win0, sem_w0)

        @pl.when(nwin > 1)
        def _():
            issue(jnp.int32(1), win1, sem_w1)

        cmax = cnt
        for pp in perms:
            cmax = jnp.maximum(cmax, lperm(cmax, pp))
        nvmax = (cmax[0] + (L - 1)) >> 4

        def walk(w, buf, carry):

            def avreg(g, carry):
                goff = pl.multiple_of(g * L, L)
                evs = [aren[j, pl.ds(goff, L)] for j in range(L)]
                ms = [(ev >> 24) == w for ev in evs]

                def hit_cond(st):
                    return jnp.any(st[0])

                def hit_body(st, ev=None):
                    m, su, pu, fk = st
                    fp = fk & 1
                    mn = jnp.where(m, lane, jnp.int32(L))
                    for pp in perms:
                        mn = jnp.minimum(mn, lperm(mn, pp))
                    mn = jnp.minimum(mn, jnp.int32(L - 1))
                    p_v = lperm(ev, mn)
                    ent = p_v[0]
                    b = ent & jnp.int32((1 << 15) - 1)
                    col = jnp.broadcast_to((ent >> 15), (L,)) & 511
                    for k in range(DIM // L):
                        stg[fp, su, pl.ds(k * L, L)] = plsc.load_gather(
                            buf, [lane + jnp.int32(k * L), col])
                    pu = jnp.where(lane == su, jnp.broadcast_to(b, (L,)),
                                   pu)

                    @pl.when(su == L - 1)
                    def _(pu=pu):
                        @pl.when(fk >= 1)
                        def _():
                            pltpu.make_async_copy(
                                stg.at[0], uv_hbm.at[six.at[0]],
                                sem_s).wait()

                        six[fp, pl.ds(0, L)] = pu
                        pltpu.async_copy(
                            stg.at[fp], uv_hbm.at[six.at[fp]], sem_s)

                    su2 = (su + 1) & (L - 1)
                    pu2 = jnp.where(su == L - 1, trash, pu)
                    fk2 = jnp.where(su == L - 1, fk + 1, fk)
                    m2 = m & (lane != mn)
                    return (m2, su2, pu2, fk2)

                su, pu, fk = carry
                for q in range(4):
                    grp = ms[4 * q]
                    for j in range(4 * q + 1, 4 * q + 4):
                        grp = grp | ms[j]

                    def qbody(su=su, pu=pu, fk=fk, q=q):
                        for j in range(4 * q, 4 * q + 4):
                            st = lax.while_loop(
                                hit_cond,
                                functools.partial(hit_body, ev=evs[j]),
                                (ms[j], su, pu, fk))
                            su, pu, fk = st[1], st[2], st[3]
                        return (su, pu, fk)

                    su, pu, fk = lax.cond(
                        jnp.any(grp), qbody,
                        lambda su=su, pu=pu, fk=fk: (su, pu, fk))
                return (su, pu, fk)

            return lax.fori_loop(0, nvmax, avreg, carry)

        def wpair(wp, carry):
            for par in range(2):
                w = wp * 2 + par
                buf, sem = wins[par], wsems[par]

                def step(carry=carry, w=w, buf=buf, sem=sem):
                    drain(w, buf, sem)
                    carry = walk(w, buf, carry)

                    @pl.when(w + 2 < nwin)
                    def _():
                        issue(w + 2, buf, sem)

                    return carry

                carry = lax.cond(w < nwin, step, lambda c=carry: c)
            return carry

        init = (jnp.int32(0), trash, jnp.int32(0))
        nwp = (nwin + 1) >> 1
        su, pu, fk = lax.fori_loop(0, nwp, wpair, init)

        @pl.when(fk >= 1)
        def _():
            pltpu.make_async_copy(
                stg.at[0], uv_hbm.at[six.at[0]], sem_s).wait()

        fp = fk & 1
        six[fp, pl.ds(0, L)] = pu
        pltpu.async_copy(stg.at[fp], uv_hbm.at[six.at[fp]], sem_s).wait()

    return sc_kernel


def _tc_score(u_ref, v_ref, r_ref, o_ref):
    u = u_ref[:, :DIM]
    v = v_ref[:, :DIM]
    rh = r_ref[:, :DIM]
    rt = r_ref[:, DIM:]
    hn = jnp.sqrt(jnp.sum(u * u, axis=1, keepdims=True))
    tn = jnp.sqrt(jnp.sum(v * v, axis=1, keepdims=True))
    un = u / jnp.maximum(hn, 1e-12)
    vn = v / jnp.maximum(tn, 1e-12)
    o_ref[...] = -jnp.sum(jnp.abs(un * rh - vn * rt), axis=1, keepdims=True)


@functools.lru_cache(maxsize=None)
def _build_tc(batch):
    blk = 512
    nblk = batch // blk
    return pl.pallas_call(
        _tc_score,
        grid=(nblk,),
        in_specs=[
            pl.BlockSpec((blk, 2 * DIM), lambda i: (i, 0)),
            pl.BlockSpec((blk, 2 * DIM), lambda i, n=nblk: (i + n, 0)),
            pl.BlockSpec((blk, 2 * DIM), lambda i: (i, 0)),
        ],
        out_specs=pl.BlockSpec((blk, 1), lambda i: (i, 0)),
        out_shape=jax.ShapeDtypeStruct((batch, 1), jnp.float32),
    )


def kernel(h, r, t, entity_emb, relation_emb):
    batch = h.shape[0]
    n_entity = entity_emb.shape[0]
    tail_base = (n_entity // TILE) * TILE
    et = entity_emb.T
    tail = lax.slice(entity_emb, (tail_base, 0), (n_entity, DIM)).T
    tail = jnp.pad(tail, ((0, 0), (0, TILE - tail.shape[1])))
    uv, ro = _build_sc(batch, n_entity)(h, r, t, et, relation_emb, tail)
    return _build_tc(batch)(uv, uv, ro)


# speedup vs baseline: 1.0298x; 1.0298x over previous
import functools

import jax
import jax.numpy as jnp
from jax import lax
from jax.experimental import pallas as pl
from jax.experimental.pallas import tpu as pltpu
from jax.experimental.pallas import tpu_sc as plsc

DIM = 64
L = 16
NC = 2
NS = 16
NW = NC * NS
TILE = 128
WTILES = 4
WIN = WTILES * TILE
RCHUNK = 128


def _cdiv(a, b):
    return (a + b - 1) // b


@functools.lru_cache(maxsize=None)
def _build_sc(batch, n_entity):
    bpw = batch // NW
    ntiles = _cdiv(n_entity, TILE)
    full_tiles = n_entity // TILE
    tail_base = full_tiles * TILE
    npieces = _cdiv(batch, 8192)
    rcap = 2 * (batch // L) + L
    mesh = plsc.VectorSubcoreMesh(core_axis_name="c", subcore_axis_name="s")

    out_types = (
        jax.ShapeDtypeStruct((2 * batch + NW, 2 * DIM), jnp.float32),
        jax.ShapeDtypeStruct((batch, 2 * DIM), jnp.float32),
    )

    @functools.partial(
        pl.kernel,
        out_type=out_types,
        mesh=mesh,
        compiler_params=pltpu.CompilerParams(needs_layout_passes=False),
        scratch_types=[
            pltpu.VMEM((DIM, WIN), jnp.float32),
            pltpu.VMEM((DIM, WIN), jnp.float32),
            pltpu.VMEM((L, rcap), jnp.int32),
            pltpu.VMEM((8192,), jnp.int32),
            pltpu.VMEM((L, 2 * DIM), jnp.float32),
            pltpu.VMEM((1, L), jnp.int32),
            pltpu.VMEM((RCHUNK, 2 * DIM), jnp.float32),
            pltpu.VMEM((bpw,), jnp.int32),
            pltpu.SemaphoreType.DMA,
            pltpu.SemaphoreType.DMA,
            pltpu.SemaphoreType.DMA,
            pltpu.SemaphoreType.DMA,
        ],
    )
    def sc_kernel(h_hbm, r_hbm, t_hbm, et_hbm, rel_hbm, tail_hbm,
                  uv_hbm, rout_hbm,
                  win0, win1, aren, piece, stg, six,
                  rel_v, ridx, sem_w0, sem_w1, sem_s, sem_r):
        wid = lax.axis_index("s") * NC + lax.axis_index("c")
        base = wid * bpw
        lane = lax.iota(jnp.int32, L)
        perms = [lane ^ k for k in (1, 2, 4, 8)]
        dnums = lax.GatherDimensionNumbers(
            offset_dims=(), collapsed_slice_dims=(0,), start_index_map=(0,))

        def lperm(v, p):
            return lax.gather(v, p[:, None], dnums, slice_sizes=(1,),
                              mode=lax.GatherScatterMode.PROMISE_IN_BOUNDS)

        pltpu.sync_copy(r_hbm.at[pl.ds(base, bpw)], ridx)
        for cc in range(bpw // RCHUNK):
            pltpu.async_copy(
                rel_hbm.at[ridx.at[pl.ds(cc * RCHUNK, RCHUNK)]], rel_v,
                sem_r).wait()
            pltpu.sync_copy(
                rel_v, rout_hbm.at[pl.ds(base + cc * RCHUNK, RCHUNK), :])

        t0 = (wid * ntiles) // NW
        t1 = ((wid + 1) * ntiles) // NW
        t1n = jnp.minimum(t1, jnp.int32(full_tiles))
        has_tail = jnp.where(t1 > t1n, jnp.int32(1), jnp.int32(0))
        nwin = (t1n - t0 + (WTILES - 1)) // WTILES + has_tail

        UNROLL = 4

        def scan_piece(p, cnt, src_hbm, tbl_bit):
            pltpu.sync_copy(src_hbm.at[pl.ds(p * 8192, 8192)], piece)

            def vstep(i, cnt):
                for s in range(UNROLL):
                    ii = i * UNROLL + s
                    v = piece[pl.ds(pl.multiple_of(ii * L, L), L)]
                    tl = v >> 7
                    m = (tl >= t0) & (tl < t1)
                    is_tl = tl >= jnp.int32(full_tiles)
                    wv = jnp.where(is_tl, nwin - 1, (tl - t0) >> 2)
                    tcv = jnp.minimum(t0 + ((tl - t0) >> 2) * WTILES,
                                      t1n - WTILES)
                    colv = jnp.where(is_tl, v - jnp.int32(tail_base),
                                     v - tcv * TILE)
                    pos = jnp.int32(p * 8192) + ii * L + lane
                    entry = (wv << 24) | (colv << 15) | tbl_bit | pos
                    entry = jnp.where(m, entry, jnp.int32(63 << 24))
                    dest = jnp.where(m, cnt, jnp.int32(rcap - L))
                    plsc.store_scatter(aren, [lane, dest], entry)
                    cnt = cnt + jnp.where(m, 1, 0)
                return cnt

            return lax.fori_loop(0, 8192 // (L * UNROLL), vstep, cnt)

        sentinel = jnp.broadcast_to(jnp.int32(63 << 24), (L,))

        def ainit(g, c):
            for j in range(L):
                aren[j, pl.ds(pl.multiple_of(g * L, L), L)] = sentinel
            return c

        lax.fori_loop(0, rcap // L, ainit, 0)

        cnt = jnp.zeros((L,), jnp.int32)
        for p in range(npieces):
            cnt = scan_piece(p, cnt, h_hbm, jnp.int32(0))
        for p in range(npieces):
            cnt = scan_piece(p, cnt, t_hbm, jnp.int32(1 << 14))

        trash = jnp.broadcast_to(jnp.int32(2 * batch), (L,)) + wid
        wins = (win0, win1)
        wsems = (sem_w0, sem_w1)

        def issue(w, buf, sem):
            is_tail = (w == nwin - 1) & (has_tail == 1)

            @pl.when(jnp.logical_not(is_tail))
            def _():
                tc = jnp.minimum(t0 + w * WTILES, t1n - WTILES)
                cb = pl.multiple_of(tc * TILE, TILE)
                pltpu.async_copy(et_hbm.at[:, pl.ds(cb, WIN)], buf, sem)

            @pl.when(is_tail)
            def _():
                pltpu.async_copy(tail_hbm, buf.at[:, pl.ds(0, TILE)], sem)

        def drain(w, buf, sem):
            is_tail = (w == nwin - 1) & (has_tail == 1)

            @pl.when(jnp.logical_not(is_tail))
            def _():
                pltpu.make_async_copy(
                    et_hbm.at[:, pl.ds(0, WIN)], buf, sem).wait()

            @pl.when(is_tail)
            def _():
                pltpu.make_async_copy(
                    tail_hbm, buf.at[:, pl.ds(0, TILE)], sem).wait()

        issue(jnp.int32(0), win0, sem_w0)

        @pl.when(nwin > 1)
        def _():
            issue(jnp.int32(1), win1, sem_w1)

        cmax = cnt
        for pp in perms:
            cmax = jnp.maximum(cmax, lperm(cmax, pp))
        nvmax = (cmax[0] + (L - 1)) >> 4

        def walk(w, buf, carry):

            def avreg(g, carry):
                goff = pl.multiple_of(g * L, L)
                evs = [aren[j, pl.ds(goff, L)] for j in range(L)]
                ms = [(ev >> 24) == w for ev in evs]

                def hit_cond(st):
                    return jnp.any(st[0])

                def hit_body(st, ev=None):
                    m, su, pu = st
                    mn = jnp.where(m, lane, jnp.int32(L))
                    for pp in perms:
                        mn = jnp.minimum(mn, lperm(mn, pp))
                    mn = jnp.minimum(mn, jnp.int32(L - 1))
                    p_v = lperm(ev, mn)
                    ent = p_v[0]
                    b = ent & jnp.int32((1 << 15) - 1)
                    col = jnp.broadcast_to((ent >> 15), (L,)) & 511
                    for k in range(DIM // L):
                        stg[su, pl.ds(k * L, L)] = plsc.load_gather(
                            buf, [lane + jnp.int32(k * L), col])
                    pu = jnp.where(lane == su, jnp.broadcast_to(b, (L,)),
                                   pu)

                    @pl.when(su == L - 1)
                    def _(pu=pu):
                        six[0, pl.ds(0, L)] = pu
                        pltpu.async_copy(
                            stg, uv_hbm.at[six.at[0]], sem_s).wait()

                    su2 = (su + 1) & (L - 1)
                    pu2 = jnp.where(su == L - 1, trash, pu)
                    m2 = m & (lane != mn)
                    return (m2, su2, pu2)

                su, pu = carry
                for q in range(4):
                    grp = ms[4 * q]
                    for j in range(4 * q + 1, 4 * q + 4):
                        grp = grp | ms[j]

                    def qbody(su=su, pu=pu, q=q):
                        for j in range(4 * q, 4 * q + 4):
                            st = lax.while_loop(
                                hit_cond,
                                functools.partial(hit_body, ev=evs[j]),
                                (ms[j], su, pu))
                            su, pu = st[1], st[2]
                        return (su, pu)

                    su, pu = lax.cond(jnp.any(grp), qbody,
                                      lambda su=su, pu=pu: (su, pu))
                return (su, pu)

            return lax.fori_loop(0, nvmax, avreg, carry)

        def wpair(wp, carry):
            for par in range(2):
                w = wp * 2 + par
                buf, sem = wins[par], wsems[par]

                def step(carry=carry, w=w, buf=buf, sem=sem):
                    drain(w, buf, sem)
                    carry = walk(w, buf, carry)

                    @pl.when(w + 2 < nwin)
                    def _():
                        issue(w + 2, buf, sem)

                    return carry

                carry = lax.cond(w < nwin, step, lambda c=carry: c)
            return carry

        init = (jnp.int32(0), trash)
        nwp = (nwin + 1) >> 1
        su, pu = lax.fori_loop(0, nwp, wpair, init)

        six[0, pl.ds(0, L)] = pu
        pltpu.async_copy(stg, uv_hbm.at[six.at[0]], sem_s).wait()

    return sc_kernel


def _tc_score(u_ref, v_ref, r_ref, o_ref):
    u = u_ref[:, :DIM]
    v = v_ref[:, :DIM]
    rh = r_ref[:, :DIM]
    rt = r_ref[:, DIM:]
    hn = jnp.sqrt(jnp.sum(u * u, axis=1, keepdims=True))
    tn = jnp.sqrt(jnp.sum(v * v, axis=1, keepdims=True))
    un = u / jnp.maximum(hn, 1e-12)
    vn = v / jnp.maximum(tn, 1e-12)
    o_ref[...] = -jnp.sum(jnp.abs(un * rh - vn * rt), axis=1, keepdims=True)


@functools.lru_cache(maxsize=None)
def _build_tc(batch):
    blk = 512
    nblk = batch // blk
    return pl.pallas_call(
        _tc_score,
        grid=(nblk,),
        in_specs=[
            pl.BlockSpec((blk, 2 * DIM), lambda i: (i, 0)),
            pl.BlockSpec((blk, 2 * DIM), lambda i, n=nblk: (i + n, 0)),
            pl.BlockSpec((blk, 2 * DIM), lambda i: (i, 0)),
        ],
        out_specs=pl.BlockSpec((blk, 1), lambda i: (i, 0)),
        out_shape=jax.ShapeDtypeStruct((batch, 1), jnp.float32),
    )


def kernel(h, r, t, entity_emb, relation_emb):
    batch = h.shape[0]
    n_entity = entity_emb.shape[0]
    tail_base = (n_entity // TILE) * TILE
    et = entity_emb.T
    tail = lax.slice(entity_emb, (tail_base, 0), (n_entity, DIM)).T
    tail = jnp.pad(tail, ((0, 0), (0, TILE - tail.shape[1])))
    uv, ro = _build_sc(batch, n_entity)(h, r, t, et, relation_emb, tail)
    return _build_tc(batch)(uv, uv, ro)
